# baseline (speedup 1.0000x reference)
import functools
import jax
import jax.numpy as jnp
from jax import lax
from jax.experimental import pallas as pl
from jax.experimental.pallas import tpu as pltpu
from jax.experimental.pallas import tpu_sc as plsc

_B = 16384
_F = 26
_V = 100000
_L = 16


def _make_kernel():
    mesh = plsc.VectorSubcoreMesh(core_axis_name="c", subcore_axis_name="s")

    @functools.partial(
        pl.kernel,
        mesh=mesh,
        out_type=jax.ShapeDtypeStruct((_B,), jnp.float32),
        scratch_types=[
            pltpu.VMEM((512,), jnp.float32),
            pltpu.SemaphoreType.DMA,
        ],
        compiler_params=pltpu.CompilerParams(needs_layout_passes=False, skip_device_barrier=True, disable_bounds_checks=True, disable_semaphore_checks=True),
    )
    def body(xt_hbm, tbl_hbm, bias_hbm, out_hbm, buf_v, sem):
        wid = lax.axis_index("s") * 2 + lax.axis_index("c")
        base = wid * 512
        pltpu.sync_copy(bias_hbm, buf_v.at[pl.ds(0, 1)])
        pltpu.sync_copy(buf_v, out_hbm.at[pl.ds(base, 512)])

    return body


_K = _make_kernel()


def kernel(X, tables, bias):
    out = _K(X, tables, bias)
    return out.reshape(_B, 1)


# speedup vs baseline: 3.9920x; 3.9920x over previous
import functools

import jax
import jax.numpy as jnp
from jax import lax
from jax.experimental import pallas as pl
from jax.experimental.pallas import tpu as pltpu
from jax.experimental.pallas import tpu_sc as plsc

_B = 16384
_F = 26
_V = 100000
_VP = 100096
_NC = 2
_NS = 16
_NW = _NC * _NS
_BPW = _B // _NW
_L = 16
_CH = 128
_NCH = _BPW // _CH


def _make_kernel():
    mesh = plsc.VectorSubcoreMesh(core_axis_name="c", subcore_axis_name="s")

    @functools.partial(
        pl.kernel,
        mesh=mesh,
        out_type=jax.ShapeDtypeStruct((_B,), jnp.float32),
        scratch_types=[
            pltpu.VMEM((_F, _BPW), jnp.int32),
            pltpu.VMEM((_F, _BPW), jnp.float32),
            pltpu.VMEM((_BPW,), jnp.float32),
            pltpu.VMEM((_L,), jnp.float32),
            pltpu.SemaphoreType.DMA,
        ],
    )
    def lr_sum(xt_hbm, tbl_hbm, bias_hbm, out_hbm, idx_v, vals_v, acc_v,
               bias_v, sem):
        wid = lax.axis_index("s") * _NC + lax.axis_index("c")
        base = wid * _BPW

        pltpu.sync_copy(xt_hbm.at[:, pl.ds(base, _BPW)], idx_v)
        pltpu.sync_copy(bias_hbm, bias_v)

        def field_body(f, _):
            off = f * _VP
            for j in range(_BPW // _L):
                sl = pl.ds(j * _L, _L)
                idx_v[f, sl] = idx_v[f, sl] + off
            for c in range(_NCH):
                sl = pl.ds(c * _CH, _CH)
                pltpu.make_async_copy(
                    tbl_hbm.at[idx_v.at[f, sl]], vals_v.at[f, sl], sem,
                ).start()
            return 0

        lax.fori_loop(0, _F, field_body, 0)

        def drain_f(f, _):
            pltpu.make_async_copy(
                tbl_hbm.at[pl.ds(0, _BPW)], vals_v.at[f], sem,
            ).wait()
            return 0

        lax.fori_loop(0, _F, drain_f, 0)

        bias_vec = bias_v[...]

        def reduce_j(j, _):
            sl = pl.ds(j * _L, _L)
            acc = bias_vec
            for f in range(_F):
                acc = acc + vals_v[f, sl]
            acc_v[sl] = acc
            return 0

        lax.fori_loop(0, _BPW // _L, reduce_j, 0)

        pltpu.sync_copy(acc_v, out_hbm.at[pl.ds(base, _BPW)])

    return lr_sum


_LR_SUM = _make_kernel()


def kernel(X, tables, bias):
    xt = X.T
    tbl = jnp.pad(tables, ((0, 0), (0, _VP - _V), (0, 0))).reshape(_F * _VP)
    bias16 = jnp.broadcast_to(bias.astype(jnp.float32), (_L,))
    out = _LR_SUM(xt, tbl, bias16)
    return out.reshape(_B, 1)


# speedup vs baseline: 8.2528x; 2.0673x over previous
import functools

import jax
import jax.numpy as jnp
from jax import lax
from jax.experimental import pallas as pl
from jax.experimental.pallas import tpu as pltpu
from jax.experimental.pallas import tpu_sc as plsc

_B = 16384
_F = 26
_V = 100000
_VP = 100096
_NC = 2
_NS = 16
_NW = _NC * _NS
_BPW = _B // _NW
_L = 16
_CH = 128
_NCH = _BPW // _CH


def _make_kernel():
    mesh = plsc.VectorSubcoreMesh(core_axis_name="c", subcore_axis_name="s")

    @functools.partial(
        pl.kernel,
        mesh=mesh,
        out_type=jax.ShapeDtypeStruct((_B,), jnp.float32),
        scratch_types=[
            pltpu.VMEM((_F, _BPW), jnp.int32),
            pltpu.VMEM((_F, _BPW), jnp.float32),
            pltpu.VMEM((_BPW,), jnp.float32),
            pltpu.VMEM((_L,), jnp.float32),
            pltpu.SemaphoreType.DMA,
        ],
    )
    def lr_sum(xt_hbm, tbl_hbm, bias_hbm, out_hbm, idx_v, vals_v, acc_v,
               bias_v, sem):
        wid = lax.axis_index("s") * _NC + lax.axis_index("c")
        base = wid * _BPW

        pltpu.sync_copy(xt_hbm.at[:, pl.ds(base, _BPW)], idx_v)
        pltpu.sync_copy(bias_hbm, bias_v)

        def field_body(f, _):
            off = f * _VP
            for j in range(_BPW // _L):
                sl = pl.ds(j * _L, _L)
                idx_v[f, sl] = idx_v[f, sl] + off
            for c in range(_NCH):
                sl = pl.ds(c * _CH, _CH)
                pltpu.make_async_copy(
                    tbl_hbm.at[idx_v.at[f, sl]], vals_v.at[f, sl], sem,
                ).start()
            return 0

        lax.fori_loop(0, _F, field_body, 0)

        def drain_f(f, _):
            pltpu.make_async_copy(
                tbl_hbm.at[pl.ds(0, _BPW)], vals_v.at[f], sem,
            ).wait()
            return 0

        lax.fori_loop(0, _F, drain_f, 0)

        bias_vec = bias_v[...]

        def reduce_j(j, _):
            sl = pl.ds(j * _L, _L)
            acc = bias_vec
            for f in range(_F):
                acc = acc + vals_v[f, sl]
            acc_v[sl] = acc
            return 0

        lax.fori_loop(0, _BPW // _L, reduce_j, 0)

        pltpu.sync_copy(acc_v, out_hbm.at[pl.ds(base, _BPW)])

    return lr_sum


_LR_SUM = _make_kernel()


def kernel(X, tables, bias):
    xt = X.T
    tbl = jnp.pad(tables[:, :, 0], ((0, 0), (0, _VP - _V))).reshape(_F * _VP)
    bias16 = jnp.broadcast_to(bias.astype(jnp.float32), (_L,))
    out = _LR_SUM(xt, tbl, bias16)
    return out.reshape(_B, 1)


# speedup vs baseline: 8.7769x; 1.0635x over previous
import functools

import jax
import jax.numpy as jnp
from jax import lax
from jax.experimental import pallas as pl
from jax.experimental.pallas import tpu as pltpu
from jax.experimental.pallas import tpu_sc as plsc

_B = 16384
_F = 26
_V = 100000
_VP = 100096
_NC = 2
_NS = 16
_NW = _NC * _NS
_BPW = _B // _NW
_L = 16
_CH = 128
_NCH = _BPW // _CH


def _make_kernel():
    mesh = plsc.VectorSubcoreMesh(core_axis_name="c", subcore_axis_name="s")

    @functools.partial(
        pl.kernel,
        mesh=mesh,
        out_type=jax.ShapeDtypeStruct((_B,), jnp.float32),
        scratch_types=[
            pltpu.VMEM((_F, _BPW), jnp.int32),
            pltpu.VMEM((_F, _BPW), jnp.float32),
            pltpu.VMEM((_BPW,), jnp.float32),
            pltpu.VMEM((_L,), jnp.float32),
            pltpu.SemaphoreType.DMA,
        ],
    )
    def lr_sum(xt_hbm, tbl_hbm, bias_hbm, out_hbm, idx_v, vals_v, acc_v,
               bias_v, sem):
        wid = lax.axis_index("s") * _NC + lax.axis_index("c")
        base = wid * _BPW

        pltpu.sync_copy(xt_hbm.at[:, pl.ds(base, _BPW)], idx_v)
        pltpu.sync_copy(bias_hbm, bias_v)

        def field_body(f, _):
            off = f * _V
            for j in range(_BPW // _L):
                sl = pl.ds(j * _L, _L)
                idx_v[f, sl] = idx_v[f, sl] + off
            for c in range(_NCH):
                sl = pl.ds(c * _CH, _CH)
                pltpu.make_async_copy(
                    tbl_hbm.at[idx_v.at[f, sl]], vals_v.at[f, sl], sem,
                ).start()
            return 0

        lax.fori_loop(0, _F, field_body, 0)

        def drain_f(f, _):
            pltpu.make_async_copy(
                tbl_hbm.at[pl.ds(0, _BPW)], vals_v.at[f], sem,
            ).wait()
            return 0

        lax.fori_loop(0, _F, drain_f, 0)

        bias_vec = bias_v[...]

        def reduce_j(j, _):
            sl = pl.ds(j * _L, _L)
            acc = bias_vec
            for f in range(_F):
                acc = acc + vals_v[f, sl]
            acc_v[sl] = acc
            return 0

        lax.fori_loop(0, _BPW // _L, reduce_j, 0)

        pltpu.sync_copy(acc_v, out_hbm.at[pl.ds(base, _BPW)])

    return lr_sum


_LR_SUM = _make_kernel()


def kernel(X, tables, bias):
    xt = X.T
    tbl = jax.lax.optimization_barrier(tables[:, :, 0]).reshape(_F * _V)
    bias16 = jnp.broadcast_to(bias.astype(jnp.float32), (_L,))
    out = _LR_SUM(xt, tbl, bias16)
    return out.reshape(_B, 1)
